# baseline (speedup 1.0000x reference)
import functools

import jax
import jax.numpy as jnp
from jax import lax
from jax.experimental import pallas as pl
from jax.experimental.pallas import tpu as pltpu
from jax.experimental.pallas import tpu_sc as plsc

B = 16384
E = 64
NC = 2
NS = 16
NW = NC * NS
BPW = B // NW

@functools.cache
def _make_gather_sc():
    mesh = plsc.VectorSubcoreMesh(core_axis_name="c", subcore_axis_name="s")

    @functools.partial(
        pl.kernel,
        mesh=mesh,
        out_type=[
            jax.ShapeDtypeStruct((B, E), jnp.float32),
            jax.ShapeDtypeStruct((B, E), jnp.float32),
        ],
        scratch_types=[
            pltpu.VMEM((BPW,), jnp.int32),
            pltpu.VMEM((BPW,), jnp.int32),
            pltpu.VMEM((BPW, E), jnp.float32),
            pltpu.VMEM((BPW, E), jnp.float32),
            pltpu.SemaphoreType.DMA,
            pltpu.SemaphoreType.DMA,
        ],
        compiler_params=pltpu.CompilerParams(use_tc_tiling_on_sc=False),
    )
    def gather_sc(user_table, movie_table, user_id, movie_id,
                  user_out, movie_out,
                  uidx_v, midx_v, urows_v, mrows_v, sem_u, sem_m):
        wid = lax.axis_index("s") * NC + lax.axis_index("c")
        base = wid * BPW
        pltpu.sync_copy(user_id.at[pl.ds(base, BPW)], uidx_v)
        pltpu.sync_copy(movie_id.at[pl.ds(base, BPW)], midx_v)
        cu = pltpu.async_copy(user_table.at[uidx_v], urows_v, sem_u)
        cm = pltpu.async_copy(movie_table.at[midx_v], mrows_v, sem_m)
        cu.wait()
        cm.wait()
        pltpu.sync_copy(urows_v, user_out.at[pl.ds(base, BPW)])
        pltpu.sync_copy(mrows_v, movie_out.at[pl.ds(base, BPW)])

    return gather_sc


def _mlp_body(ue_ref, me_ref, w1a_ref, w1b_ref, b1_ref, w2_ref, b2_ref,
              w3_ref, b3_ref, out_ref):
    h = (jnp.dot(ue_ref[...], w1a_ref[...], preferred_element_type=jnp.float32)
         + jnp.dot(me_ref[...], w1b_ref[...], preferred_element_type=jnp.float32)
         + b1_ref[...])
    h = jnp.maximum(h, 0.0)
    h = jnp.dot(h, w2_ref[...], preferred_element_type=jnp.float32) + b2_ref[...]
    h = jnp.maximum(h, 0.0)
    o = jnp.dot(h, w3_ref[...], preferred_element_type=jnp.float32) + b3_ref[...]
    out_ref[...] = jax.nn.sigmoid(o)


BB = 2048


def _mlp_tc(ue, me, w1a, w1b, b1, w2, b2, w3, b3):
    fixed = lambda i: (0, 0)
    return pl.pallas_call(
        _mlp_body,
        grid=(B // BB,),
        in_specs=[
            pl.BlockSpec((BB, E), lambda i: (i, 0)),
            pl.BlockSpec((BB, E), lambda i: (i, 0)),
            pl.BlockSpec((E, 64), fixed),
            pl.BlockSpec((E, 64), fixed),
            pl.BlockSpec((1, 64), fixed),
            pl.BlockSpec((64, 32), fixed),
            pl.BlockSpec((1, 32), fixed),
            pl.BlockSpec((32, 1), fixed),
            pl.BlockSpec((1, 1), fixed),
        ],
        out_specs=pl.BlockSpec((BB, 1), lambda i: (i, 0)),
        out_shape=jax.ShapeDtypeStruct((B, 1), jnp.float32),
    )(ue, me, w1a, w1b, b1, w2, b2, w3, b3)


def kernel(user_id, movie_id, user_table, movie_table, W1, b1, W2, b2, W3, b3):
    user_emb, movie_emb = _make_gather_sc()(user_table, movie_table,
                                     user_id.astype(jnp.int32),
                                     movie_id.astype(jnp.int32))
    return _mlp_tc(user_emb, movie_emb,
                   W1[:E], W1[E:], b1.reshape(1, 64),
                   W2, b2.reshape(1, 32), W3, b3.reshape(1, 1))


# speedup vs baseline: 1.1321x; 1.1321x over previous
import functools

import jax
import jax.numpy as jnp
from jax import lax
from jax.experimental import pallas as pl
from jax.experimental.pallas import tpu as pltpu
from jax.experimental.pallas import tpu_sc as plsc

B = 16384
E = 64
NC = 2
NS = 16
NW = NC * NS

USLAB_BIG = 245 * 128
USLAB_SM = 244 * 128
UTAIL_LO = 999936
UTAIL_N = 64
MSLAB_BIG = 25 * 128
MSLAB_SM = 24 * 128
MTAIL_LO = 99968
MTAIL_N = 32

PBITS = 15
PMASK = (1 << PBITS) - 1
HB = 512
DUMMY = B
CW = 31 * 128


def _wid():
    return lax.axis_index("c") * NS + lax.axis_index("s")


def _mesh():
    return plsc.VectorSubcoreMesh(core_axis_name="c", subcore_axis_name="s")


@functools.cache
def _make_select_sc():
    @functools.partial(
        pl.kernel,
        mesh=_mesh(),
        out_type=[
            jax.ShapeDtypeStruct((NW * B,), jnp.int32),
            jax.ShapeDtypeStruct((NW * B,), jnp.int32),
            jax.ShapeDtypeStruct((NW * B,), jnp.int32),
            jax.ShapeDtypeStruct((NW * B,), jnp.int32),
            jax.ShapeDtypeStruct((2 * NW * 16,), jnp.int32),
        ],
        scratch_types=[
            pltpu.VMEM((B,), jnp.int32),
            pltpu.VMEM((B,), jnp.int32),
            pltpu.VMEM((2048,), jnp.int32),
            pltpu.VMEM((16,), jnp.int32),
        ],
        compiler_params=pltpu.CompilerParams(needs_layout_passes=False),
    )
    def select_sc(user_id, movie_id, hup, hub, hmp, hmb, cnts,
                  hitp_v, hitb_v, idseg_v, cnt_v):
        wid = _wid()
        iota16 = lax.iota(jnp.int32, 16)

        def phase(ids_hbm, rlo, wbig, wsm, nbig,
                  tail_owner, tail_lo, tail_n, hp_out, hb_out, cslot):
            slab_w = jnp.where(wid < nbig, wbig, wsm)
            rhi = rlo + slab_w
            tail_off = wbig

            def seg(si, carry):
                pltpu.sync_copy(ids_hbm.at[pl.ds(si * 2048, 2048)], idseg_v)

                def grp(g, carry2):
                    n3, nvec = carry2
                    v = idseg_v[pl.ds(g * 16, 16)]
                    bvec = iota16 + (si * 2048 + g * 16)
                    m = (v >= rlo) & (v < rhi)
                    p = v - rlo
                    mx = ((wid == tail_owner) & (v >= tail_lo)
                          & (v < tail_lo + tail_n))
                    p = jnp.where(mx, v - tail_lo + tail_off, p)
                    m = m | mx
                    plsc.store_compressed(hitp_v.at[pl.ds(n3, 16)],
                                          p, mask=m)
                    plsc.store_compressed(hitb_v.at[pl.ds(n3, 16)],
                                          bvec, mask=m)
                    ca = plsc.all_reduce_population_count(m)
                    return (n3 + lax.reduce_max(ca, (0,)), nvec + ca)

                return lax.fori_loop(0, 128, grp, carry)

            _, nvec = lax.fori_loop(0, B // 2048, seg,
                                    (0, jnp.zeros((16,), jnp.int32)))
            cnt_v[pl.ds(0, 16)] = nvec
            pltpu.sync_copy(hitp_v, hp_out.at[pl.ds(wid * B, B)])
            pltpu.sync_copy(hitb_v, hb_out.at[pl.ds(wid * B, B)])
            pltpu.sync_copy(cnt_v,
                            cnts.at[pl.ds(cslot * NW * 16 + wid * 16, 16)])

        rlo_u = wid * USLAB_SM + jnp.minimum(wid, 4) * 128
        phase(user_id, rlo_u, USLAB_BIG, USLAB_SM, 4, 0,
              UTAIL_LO, UTAIL_N, hup, hub, 0)
        rlo_m = wid * MSLAB_SM + jnp.minimum(wid, 13) * 128
        phase(movie_id, rlo_m, MSLAB_BIG, MSLAB_SM, 13, 1,
              MTAIL_LO, MTAIL_N, hmp, hmb, 1)

    return select_sc


@functools.cache
def _make_scan_sc(np_static):
    @functools.partial(
        pl.kernel,
        mesh=_mesh(),
        out_type=[jax.ShapeDtypeStruct((B + 16, 128), jnp.float32)
                  for _ in range(4)],
        scratch_types=[
            pltpu.VMEM((8, CW), jnp.float32),
            pltpu.VMEM((HB, 128), jnp.float32),
            pltpu.VMEM((HB,), jnp.int32),
            pltpu.VMEM((HB,), jnp.int32),
            pltpu.VMEM((HB // 128, 128), jnp.int32),
            pltpu.VMEM((UTAIL_N, 128), jnp.float32),
            pltpu.VMEM((16,), jnp.int32),
        ],
        compiler_params=pltpu.CompilerParams(needs_layout_passes=False),
    )
    def scan_sc(utT, mtT, utail, mtail, hup, hub, hmp, hmb, cnts,
                ua, ub, ma, mb,
                chunk_v, rows_v, hitp_v, hitb_v, stage_v, tail_v, cnt_v):
        c = lax.axis_index("c")
        wid = _wid()
        iota16 = lax.iota(jnp.int32, 16)

        def phase(tT, rlo, nchunks, clamp, hp_hbm, hb_hbm, cslot,
                  tail_hbm, tail_owner, tail_off, tail_n, out_a, out_b):
            pltpu.sync_copy(
                cnts.at[pl.ds(cslot * NW * 16 + wid * 16, 16)], cnt_v)
            nvec = cnt_v[pl.ds(0, 16)]

            @pl.when(wid == tail_owner)
            def _():
                pltpu.sync_copy(tail_hbm.at[pl.ds(0, tail_n)],
                                tail_v.at[pl.ds(0, tail_n)])

            def extract_groups(hbase, clo_local, w, gather_fn):
                def grp(g, cc):
                    kv = iota16 + g * 16
                    valid = (kv + hbase) < nvec
                    p = hitp_v[pl.ds(g * 16, 16)] - clo_local
                    mc = valid & (p >= 0) & (p < w)
                    psafe = jnp.where(mc, p, 0)
                    gather_fn(kv, psafe, mc)
                    return cc

                lax.fori_loop(0, HB // 16, grp, 0)

            def one_pass(h, _unused):
                hbase = h * HB
                pltpu.sync_copy(hp_hbm.at[pl.ds(wid * B + hbase, HB)],
                                hitp_v)
                pltpu.sync_copy(hb_hbm.at[pl.ds(wid * B + hbase, HB)],
                                hitb_v)

                def chunk_octet(q, cc):
                    ci = lax.shift_right_logical(q, 3)
                    a = lax.bitwise_and(q, 7)
                    clo = jnp.minimum(rlo + ci * CW, clamp)
                    pltpu.sync_copy(
                        tT.at[pl.ds(pl.multiple_of(a * 8, 8), 8),
                              pl.ds(pl.multiple_of(clo, 128), CW)],
                        chunk_v)

                    def gfn(kv, psafe, mc):
                        for jj in range(8):
                            x = plsc.load_gather(
                                chunk_v,
                                [jnp.broadcast_to(jj, (16,)), psafe],
                                mask=mc)
                            plsc.store_scatter(
                                rows_v,
                                [kv, jnp.broadcast_to(a * 8 + jj, (16,))],
                                x, mask=mc)

                    extract_groups(hbase, clo - rlo, CW, gfn)
                    return cc

                lax.fori_loop(0, nchunks * 8, chunk_octet, 0)

                @pl.when(wid == tail_owner)
                def _():
                    def gfn(kv, psafe, mc):
                        for j in range(E):
                            x = plsc.load_gather(
                                tail_v,
                                [psafe, jnp.broadcast_to(j, (16,))], mask=mc)
                            plsc.store_scatter(
                                rows_v,
                                [kv, jnp.broadcast_to(j, (16,))], x, mask=mc)

                    extract_groups(hbase, tail_off, tail_n, gfn)

                def sgrp(g, cc):
                    kv = iota16 + g * 16
                    valid = (kv + hbase) < nvec
                    bvec = jnp.where(valid, hitb_v[pl.ds(g * 16, 16)],
                                     DUMMY + iota16)
                    stage_v[lax.shift_right_logical(g, 3),
                            pl.ds(lax.bitwise_and(g, 7) * 16, 16)] = bvec
                    return cc

                lax.fori_loop(0, HB // 16, sgrp, 0)
                for q in range(HB // 128):
                    @pl.when(c == 0)
                    def _(q=q):
                        pltpu.sync_copy(rows_v.at[pl.ds(q * 128, 128)],
                                        out_a.at[stage_v.at[q]])

                    @pl.when(c == 1)
                    def _(q=q):
                        pltpu.sync_copy(rows_v.at[pl.ds(q * 128, 128)],
                                        out_b.at[stage_v.at[q]])
                return _unused

            lax.fori_loop(0, np_static, one_pass, 0)

        rlo_u = wid * USLAB_SM + jnp.minimum(wid, 4) * 128
        phase(utT, rlo_u, 8, 999936 - CW, hup, hub, 0,
              utail, 0, USLAB_BIG, UTAIL_N, ua, ub)
        rlo_m = wid * MSLAB_SM + jnp.minimum(wid, 13) * 128
        phase(mtT, rlo_m, 1, 99968 - CW, hmp, hmb, 1,
              mtail, 1, MSLAB_BIG, MTAIL_N, ma, mb)

    return scan_sc


def _mlp_body(ua_ref, ub_ref, ma_ref, mb_ref, uid_ref, mid_ref,
              w1a_ref, w1b_ref, b1_ref, w2_ref, b2_ref,
              w3_ref, b3_ref, out_ref):
    ua_hi = 16 * USLAB_SM + 4 * 128
    ma_hi = 16 * MSLAB_SM + 13 * 128
    use_a_u = (uid_ref[...] < ua_hi) | (uid_ref[...] >= UTAIL_LO)
    use_a_m = (mid_ref[...] < ma_hi) | (mid_ref[...] >= MTAIL_LO)
    ue = jnp.where(use_a_u, ua_ref[...][:, :E], ub_ref[...][:, :E])
    me = jnp.where(use_a_m, ma_ref[...][:, :E], mb_ref[...][:, :E])
    h = (jnp.dot(ue, w1a_ref[...], preferred_element_type=jnp.float32)
         + jnp.dot(me, w1b_ref[...], preferred_element_type=jnp.float32)
         + b1_ref[...])
    h = jnp.maximum(h, 0.0)
    h = jnp.dot(h, w2_ref[...], preferred_element_type=jnp.float32) + b2_ref[...]
    h = jnp.maximum(h, 0.0)
    o = jnp.dot(h, w3_ref[...], preferred_element_type=jnp.float32) + b3_ref[...]
    out_ref[...] = jax.nn.sigmoid(o)


BB = 2048


def _mlp_tc(ua, ub, ma, mb, uid2, mid2, w1a, w1b, b1, w2, b2, w3, b3):
    fixed = lambda i: (0, 0)
    emb = lambda i: (i, 0)
    return pl.pallas_call(
        _mlp_body,
        grid=(B // BB,),
        in_specs=[
            pl.BlockSpec((BB, 128), emb),
            pl.BlockSpec((BB, 128), emb),
            pl.BlockSpec((BB, 128), emb),
            pl.BlockSpec((BB, 128), emb),
            pl.BlockSpec((BB, 1), emb),
            pl.BlockSpec((BB, 1), emb),
            pl.BlockSpec((E, 64), fixed),
            pl.BlockSpec((E, 64), fixed),
            pl.BlockSpec((1, 64), fixed),
            pl.BlockSpec((64, 32), fixed),
            pl.BlockSpec((1, 32), fixed),
            pl.BlockSpec((32, 1), fixed),
            pl.BlockSpec((1, 1), fixed),
        ],
        out_specs=pl.BlockSpec((BB, 1), emb),
        out_shape=jax.ShapeDtypeStruct((B, 1), jnp.float32),
    )(ua, ub, ma, mb, uid2, mid2, w1a, w1b, b1, w2, b2, w3, b3)


def kernel(user_id, movie_id, user_table, movie_table, W1, b1, W2, b2, W3, b3):
    utT = user_table.T
    mtT = movie_table.T
    uid = user_id.astype(jnp.int32)
    mid = movie_id.astype(jnp.int32)
    utail = jnp.pad(user_table[UTAIL_LO:, :], ((0, 0), (0, 128 - E)))
    mtail = jnp.pad(movie_table[MTAIL_LO:, :], ((0, 0), (0, 128 - E)))
    hup, hub, hmp, hmb, cnts = _make_select_sc()(uid, mid)
    counts = cnts.reshape(2 * NW, 16)[:, 0]
    overflow = jnp.max(counts) > 2 * HB
    scan_args = (utT, mtT, utail, mtail, hup, hub, hmp, hmb, cnts)
    ua, ub, ma, mb = lax.cond(
        overflow,
        lambda: _make_scan_sc(B // HB)(*scan_args),
        lambda: _make_scan_sc(2)(*scan_args),
    )
    return _mlp_tc(ua, ub, ma, mb,
                   uid.reshape(B, 1), mid.reshape(B, 1),
                   W1[:E], W1[E:], b1.reshape(1, 64),
                   W2, b2.reshape(1, 32), W3, b3.reshape(1, 1))


# speedup vs baseline: 1.7882x; 1.5796x over previous
import functools

import jax
import jax.numpy as jnp
from jax import lax
from jax.experimental import pallas as pl
from jax.experimental.pallas import tpu as pltpu
from jax.experimental.pallas import tpu_sc as plsc

B = 16384
E = 64
NC = 2
NS = 16
NW = NC * NS

USLAB_BIG = 245 * 128
USLAB_SM = 244 * 128
UTAIL_LO = 999936
UTAIL_N = 64
MSLAB_BIG = 25 * 128
MSLAB_SM = 24 * 128
MTAIL_LO = 99968
MTAIL_N = 32

PBITS = 15
PMASK = (1 << PBITS) - 1
HB = 640
HSTRIDE = 26 * HB
DUMMY = B
CW = 31 * 128


def _wid():
    return lax.axis_index("c") * NS + lax.axis_index("s")


def _mesh():
    return plsc.VectorSubcoreMesh(core_axis_name="c", subcore_axis_name="s")


@functools.cache
def _make_select_sc():
    @functools.partial(
        pl.kernel,
        mesh=_mesh(),
        out_type=[
            jax.ShapeDtypeStruct((NW * HSTRIDE,), jnp.int32),
            jax.ShapeDtypeStruct((NW * HSTRIDE,), jnp.int32),
            jax.ShapeDtypeStruct((NW * HSTRIDE,), jnp.int32),
            jax.ShapeDtypeStruct((NW * HSTRIDE,), jnp.int32),
            jax.ShapeDtypeStruct((2 * NW * 16,), jnp.int32),
        ],
        scratch_types=[
            pltpu.VMEM((B,), jnp.int32),
            pltpu.VMEM((B,), jnp.int32),
            pltpu.VMEM((2048,), jnp.int32),
            pltpu.VMEM((16,), jnp.int32),
        ],
        compiler_params=pltpu.CompilerParams(needs_layout_passes=False),
    )
    def select_sc(user_id, movie_id, hup, hub, hmp, hmb, cnts,
                  hitp_v, hitb_v, idseg_v, cnt_v):
        wid = _wid()
        iota16 = lax.iota(jnp.int32, 16)

        def phase(ids_hbm, rlo, wbig, wsm, nbig,
                  tail_owner, tail_lo, tail_n, hp_out, hb_out, cslot):
            slab_w = jnp.where(wid < nbig, wbig, wsm)
            rhi = rlo + slab_w
            tail_off = wbig

            def seg(si, carry):
                pltpu.sync_copy(ids_hbm.at[pl.ds(si * 2048, 2048)], idseg_v)

                def grp(g, carry2):
                    n3, nvec = carry2
                    v = idseg_v[pl.ds(g * 16, 16)]
                    bvec = iota16 + (si * 2048 + g * 16)
                    m = (v >= rlo) & (v < rhi)
                    p = v - rlo
                    mx = ((wid == tail_owner) & (v >= tail_lo)
                          & (v < tail_lo + tail_n))
                    p = jnp.where(mx, v - tail_lo + tail_off, p)
                    m = m | mx
                    plsc.store_compressed(hitp_v.at[pl.ds(n3, 16)],
                                          p, mask=m)
                    plsc.store_compressed(hitb_v.at[pl.ds(n3, 16)],
                                          bvec, mask=m)
                    ca = plsc.all_reduce_population_count(m)
                    return (n3 + lax.reduce_max(ca, (0,)), nvec + ca)

                return lax.fori_loop(0, 128, grp, carry)

            _, nvec = lax.fori_loop(0, B // 2048, seg,
                                    (0, jnp.zeros((16,), jnp.int32)))
            cnt_v[pl.ds(0, 16)] = nvec
            pltpu.sync_copy(hitp_v, hp_out.at[pl.ds(wid * HSTRIDE, B)])
            pltpu.sync_copy(hitb_v, hb_out.at[pl.ds(wid * HSTRIDE, B)])
            pltpu.sync_copy(cnt_v,
                            cnts.at[pl.ds(cslot * NW * 16 + wid * 16, 16)])

        rlo_u = wid * USLAB_SM + jnp.minimum(wid, 4) * 128
        phase(user_id, rlo_u, USLAB_BIG, USLAB_SM, 4, 0,
              UTAIL_LO, UTAIL_N, hup, hub, 0)
        rlo_m = wid * MSLAB_SM + jnp.minimum(wid, 13) * 128
        phase(movie_id, rlo_m, MSLAB_BIG, MSLAB_SM, 13, 1,
              MTAIL_LO, MTAIL_N, hmp, hmb, 1)

    return select_sc


@functools.cache
def _make_scan_sc(np_static):
    @functools.partial(
        pl.kernel,
        mesh=_mesh(),
        out_type=[jax.ShapeDtypeStruct((B + 16, 128), jnp.float32)
                  for _ in range(4)],
        scratch_types=[
            pltpu.VMEM((8, CW), jnp.float32),
            pltpu.VMEM((HB, 128), jnp.float32),
            pltpu.VMEM((HB,), jnp.int32),
            pltpu.VMEM((HB,), jnp.int32),
            pltpu.VMEM((HB // 128, 128), jnp.int32),
            pltpu.VMEM((UTAIL_N, 128), jnp.float32),
            pltpu.VMEM((16,), jnp.int32),
        ],
        compiler_params=pltpu.CompilerParams(needs_layout_passes=False),
    )
    def scan_sc(utT, mtT, utail, mtail, hup, hub, hmp, hmb, cnts,
                ua, ub, ma, mb,
                chunk_v, rows_v, hitp_v, hitb_v, stage_v, tail_v, cnt_v):
        c = lax.axis_index("c")
        wid = _wid()
        iota16 = lax.iota(jnp.int32, 16)

        def phase(tT, rlo, nchunks, clamp, hp_hbm, hb_hbm, cslot,
                  tail_hbm, tail_owner, tail_off, tail_n, out_a, out_b):
            pltpu.sync_copy(
                cnts.at[pl.ds(cslot * NW * 16 + wid * 16, 16)], cnt_v)
            nvec = cnt_v[pl.ds(0, 16)]

            @pl.when(wid == tail_owner)
            def _():
                pltpu.sync_copy(tail_hbm.at[pl.ds(0, tail_n)],
                                tail_v.at[pl.ds(0, tail_n)])

            def extract_groups(hbase, clo_local, w, gather_fn):
                def grp(g, cc):
                    kv = iota16 + g * 16
                    valid = (kv + hbase) < nvec
                    p = hitp_v[pl.ds(g * 16, 16)] - clo_local
                    mc = valid & (p >= 0) & (p < w)
                    psafe = jnp.where(mc, p, 0)
                    gather_fn(kv, psafe, mc)
                    return cc

                lax.fori_loop(0, HB // 16, grp, 0)

            def one_pass(h, _unused):
                hbase = h * HB
                pltpu.sync_copy(hp_hbm.at[pl.ds(wid * HSTRIDE + hbase, HB)],
                                hitp_v)
                pltpu.sync_copy(hb_hbm.at[pl.ds(wid * HSTRIDE + hbase, HB)],
                                hitb_v)

                def chunk_octet(q, cc):
                    ci = lax.shift_right_logical(q, 3)
                    a = lax.bitwise_and(q, 7)
                    clo = jnp.minimum(rlo + ci * CW, clamp)
                    pltpu.sync_copy(
                        tT.at[pl.ds(pl.multiple_of(a * 8, 8), 8),
                              pl.ds(pl.multiple_of(clo, 128), CW)],
                        chunk_v)

                    def gfn(kv, psafe, mc):
                        for jj in range(8):
                            x = plsc.load_gather(
                                chunk_v,
                                [jnp.broadcast_to(jj, (16,)), psafe],
                                mask=mc)
                            plsc.store_scatter(
                                rows_v,
                                [kv, jnp.broadcast_to(a * 8 + jj, (16,))],
                                x, mask=mc)

                    extract_groups(hbase, clo - rlo, CW, gfn)
                    return cc

                lax.fori_loop(0, nchunks * 8, chunk_octet, 0)

                @pl.when(wid == tail_owner)
                def _():
                    def gfn(kv, psafe, mc):
                        for j in range(E):
                            x = plsc.load_gather(
                                tail_v,
                                [psafe, jnp.broadcast_to(j, (16,))], mask=mc)
                            plsc.store_scatter(
                                rows_v,
                                [kv, jnp.broadcast_to(j, (16,))], x, mask=mc)

                    extract_groups(hbase, tail_off, tail_n, gfn)

                def sgrp(g, cc):
                    kv = iota16 + g * 16
                    valid = (kv + hbase) < nvec
                    bvec = jnp.where(valid, hitb_v[pl.ds(g * 16, 16)],
                                     DUMMY + iota16)
                    stage_v[lax.shift_right_logical(g, 3),
                            pl.ds(lax.bitwise_and(g, 7) * 16, 16)] = bvec
                    return cc

                lax.fori_loop(0, HB // 16, sgrp, 0)
                for q in range(HB // 128):
                    @pl.when(c == 0)
                    def _(q=q):
                        pltpu.sync_copy(rows_v.at[pl.ds(q * 128, 128)],
                                        out_a.at[stage_v.at[q]])

                    @pl.when(c == 1)
                    def _(q=q):
                        pltpu.sync_copy(rows_v.at[pl.ds(q * 128, 128)],
                                        out_b.at[stage_v.at[q]])
                return _unused

            lax.fori_loop(0, np_static, one_pass, 0)

        rlo_u = wid * USLAB_SM + jnp.minimum(wid, 4) * 128
        phase(utT, rlo_u, 8, 999936 - CW, hup, hub, 0,
              utail, 0, USLAB_BIG, UTAIL_N, ua, ub)
        rlo_m = wid * MSLAB_SM + jnp.minimum(wid, 13) * 128
        phase(mtT, rlo_m, 1, 99968 - CW, hmp, hmb, 1,
              mtail, 1, MSLAB_BIG, MTAIL_N, ma, mb)

    return scan_sc


def _mlp_body(ua_ref, ub_ref, ma_ref, mb_ref, uid_ref, mid_ref,
              w1a_ref, w1b_ref, b1_ref, w2_ref, b2_ref,
              w3_ref, b3_ref, out_ref):
    ua_hi = 16 * USLAB_SM + 4 * 128
    ma_hi = 16 * MSLAB_SM + 13 * 128
    use_a_u = (uid_ref[...] < ua_hi) | (uid_ref[...] >= UTAIL_LO)
    use_a_m = (mid_ref[...] < ma_hi) | (mid_ref[...] >= MTAIL_LO)
    ue = jnp.where(use_a_u, ua_ref[...][:, :E], ub_ref[...][:, :E])
    me = jnp.where(use_a_m, ma_ref[...][:, :E], mb_ref[...][:, :E])
    h = (jnp.dot(ue, w1a_ref[...], preferred_element_type=jnp.float32)
         + jnp.dot(me, w1b_ref[...], preferred_element_type=jnp.float32)
         + b1_ref[...])
    h = jnp.maximum(h, 0.0)
    h = jnp.dot(h, w2_ref[...], preferred_element_type=jnp.float32) + b2_ref[...]
    h = jnp.maximum(h, 0.0)
    o = jnp.dot(h, w3_ref[...], preferred_element_type=jnp.float32) + b3_ref[...]
    out_ref[...] = jax.nn.sigmoid(o)


BB = 2048


def _mlp_tc(ua, ub, ma, mb, uid2, mid2, w1a, w1b, b1, w2, b2, w3, b3):
    fixed = lambda i: (0, 0)
    emb = lambda i: (i, 0)
    return pl.pallas_call(
        _mlp_body,
        grid=(B // BB,),
        in_specs=[
            pl.BlockSpec((BB, 128), emb),
            pl.BlockSpec((BB, 128), emb),
            pl.BlockSpec((BB, 128), emb),
            pl.BlockSpec((BB, 128), emb),
            pl.BlockSpec((BB, 1), emb),
            pl.BlockSpec((BB, 1), emb),
            pl.BlockSpec((E, 64), fixed),
            pl.BlockSpec((E, 64), fixed),
            pl.BlockSpec((1, 64), fixed),
            pl.BlockSpec((64, 32), fixed),
            pl.BlockSpec((1, 32), fixed),
            pl.BlockSpec((32, 1), fixed),
            pl.BlockSpec((1, 1), fixed),
        ],
        out_specs=pl.BlockSpec((BB, 1), emb),
        out_shape=jax.ShapeDtypeStruct((B, 1), jnp.float32),
    )(ua, ub, ma, mb, uid2, mid2, w1a, w1b, b1, w2, b2, w3, b3)


def kernel(user_id, movie_id, user_table, movie_table, W1, b1, W2, b2, W3, b3):
    utT = user_table.T
    mtT = movie_table.T
    uid = user_id.astype(jnp.int32)
    mid = movie_id.astype(jnp.int32)
    utail = jnp.pad(user_table[UTAIL_LO:, :], ((0, 0), (0, 128 - E)))
    mtail = jnp.pad(movie_table[MTAIL_LO:, :], ((0, 0), (0, 128 - E)))
    hup, hub, hmp, hmb, cnts = _make_select_sc()(uid, mid)
    counts = cnts.reshape(2 * NW, 16)[:, 0]
    overflow = jnp.max(counts) > HB
    scan_args = (utT, mtT, utail, mtail, hup, hub, hmp, hmb, cnts)
    ua, ub, ma, mb = lax.cond(
        overflow,
        lambda: _make_scan_sc(26)(*scan_args),
        lambda: _make_scan_sc(1)(*scan_args),
    )
    return _mlp_tc(ua, ub, ma, mb,
                   uid.reshape(B, 1), mid.reshape(B, 1),
                   W1[:E], W1[E:], b1.reshape(1, 64),
                   W2, b2.reshape(1, 32), W3, b3.reshape(1, 1))


# speedup vs baseline: 1.9276x; 1.0780x over previous
import functools

import jax
import jax.numpy as jnp
from jax import lax
from jax.experimental import pallas as pl
from jax.experimental.pallas import tpu as pltpu
from jax.experimental.pallas import tpu_sc as plsc

B = 16384
E = 64
NC = 2
NS = 16
NW = NC * NS

USLAB_BIG = 245 * 128
USLAB_SM = 244 * 128
UTAIL_LO = 999936
UTAIL_N = 64
MSLAB_BIG = 25 * 128
MSLAB_SM = 24 * 128
MTAIL_LO = 99968
MTAIL_N = 32

PBITS = 15
PMASK = (1 << PBITS) - 1
HB = 640
HSTRIDE = 26 * HB
DUMMY = B
CW = 17 * 128


def _wid():
    return lax.axis_index("c") * NS + lax.axis_index("s")


def _mesh():
    return plsc.VectorSubcoreMesh(core_axis_name="c", subcore_axis_name="s")


@functools.cache
def _make_select_sc():
    @functools.partial(
        pl.kernel,
        mesh=_mesh(),
        out_type=[
            jax.ShapeDtypeStruct((NW * HSTRIDE,), jnp.int32),
            jax.ShapeDtypeStruct((NW * HSTRIDE,), jnp.int32),
            jax.ShapeDtypeStruct((NW * HSTRIDE,), jnp.int32),
            jax.ShapeDtypeStruct((NW * HSTRIDE,), jnp.int32),
            jax.ShapeDtypeStruct((2 * NW * 16,), jnp.int32),
        ],
        scratch_types=[
            pltpu.VMEM((B,), jnp.int32),
            pltpu.VMEM((B,), jnp.int32),
            pltpu.VMEM((2048,), jnp.int32),
            pltpu.VMEM((16,), jnp.int32),
        ],
        compiler_params=pltpu.CompilerParams(needs_layout_passes=False),
    )
    def select_sc(user_id, movie_id, hup, hub, hmp, hmb, cnts,
                  hitp_v, hitb_v, idseg_v, cnt_v):
        wid = _wid()
        iota16 = lax.iota(jnp.int32, 16)

        def phase(ids_hbm, rlo, wbig, wsm, nbig,
                  tail_owner, tail_lo, tail_n, hp_out, hb_out, cslot):
            slab_w = jnp.where(wid < nbig, wbig, wsm)
            rhi = rlo + slab_w
            tail_off = wbig

            def seg(si, carry):
                pltpu.sync_copy(ids_hbm.at[pl.ds(si * 2048, 2048)], idseg_v)

                def grp(g, carry2):
                    n3, nvec = carry2
                    v = idseg_v[pl.ds(g * 16, 16)]
                    bvec = iota16 + (si * 2048 + g * 16)
                    m = (v >= rlo) & (v < rhi)
                    p = v - rlo
                    mx = ((wid == tail_owner) & (v >= tail_lo)
                          & (v < tail_lo + tail_n))
                    p = jnp.where(mx, v - tail_lo + tail_off, p)
                    m = m | mx
                    plsc.store_compressed(hitp_v.at[pl.ds(n3, 16)],
                                          p, mask=m)
                    plsc.store_compressed(hitb_v.at[pl.ds(n3, 16)],
                                          bvec, mask=m)
                    ca = plsc.all_reduce_population_count(m)
                    return (n3 + lax.reduce_max(ca, (0,)), nvec + ca)

                return lax.fori_loop(0, 128, grp, carry)

            _, nvec = lax.fori_loop(0, B // 2048, seg,
                                    (0, jnp.zeros((16,), jnp.int32)))
            cnt_v[pl.ds(0, 16)] = nvec
            pltpu.sync_copy(hitp_v, hp_out.at[pl.ds(wid * HSTRIDE, B)])
            pltpu.sync_copy(hitb_v, hb_out.at[pl.ds(wid * HSTRIDE, B)])
            pltpu.sync_copy(cnt_v,
                            cnts.at[pl.ds(cslot * NW * 16 + wid * 16, 16)])

        rlo_u = wid * USLAB_SM + jnp.minimum(wid, 4) * 128
        phase(user_id, rlo_u, USLAB_BIG, USLAB_SM, 4, 0,
              UTAIL_LO, UTAIL_N, hup, hub, 0)
        rlo_m = wid * MSLAB_SM + jnp.minimum(wid, 13) * 128
        phase(movie_id, rlo_m, MSLAB_BIG, MSLAB_SM, 13, 1,
              MTAIL_LO, MTAIL_N, hmp, hmb, 1)

    return select_sc


@functools.cache
def _make_scan_sc(np_static):
    @functools.partial(
        pl.kernel,
        mesh=_mesh(),
        out_type=[jax.ShapeDtypeStruct((B + 16, 128), jnp.float32)
                  for _ in range(4)],
        scratch_types=[
            pltpu.VMEM((2, 8, CW), jnp.float32),
            pltpu.VMEM((HB, 128), jnp.float32),
            pltpu.VMEM((HB,), jnp.int32),
            pltpu.VMEM((HB,), jnp.int32),
            pltpu.VMEM((HB // 128, 128), jnp.int32),
            pltpu.VMEM((UTAIL_N, 128), jnp.float32),
            pltpu.VMEM((16,), jnp.int32),
            pltpu.SemaphoreType.DMA,
            pltpu.SemaphoreType.DMA,
        ],
        compiler_params=pltpu.CompilerParams(needs_layout_passes=False),
    )
    def scan_sc(utT, mtT, utail, mtail, hup, hub, hmp, hmb, cnts,
                ua, ub, ma, mb,
                chunk_v, rows_v, hitp_v, hitb_v, stage_v, tail_v, cnt_v,
                semA, semB):
        c = lax.axis_index("c")
        wid = _wid()
        iota16 = lax.iota(jnp.int32, 16)

        def phase(tT, rlo, nchunks, clamp, hp_hbm, hb_hbm, cslot,
                  tail_hbm, tail_owner, tail_off, tail_n, out_a, out_b):
            pltpu.sync_copy(
                cnts.at[pl.ds(cslot * NW * 16 + wid * 16, 16)], cnt_v)
            nvec = cnt_v[pl.ds(0, 16)]

            @pl.when(wid == tail_owner)
            def _():
                pltpu.sync_copy(tail_hbm.at[pl.ds(0, tail_n)],
                                tail_v.at[pl.ds(0, tail_n)])

            def extract_groups(hbase, clo_local, w, gather_fn):
                def grp(g, cc):
                    kv = iota16 + g * 16
                    valid = (kv + hbase) < nvec
                    p = hitp_v[pl.ds(g * 16, 16)] - clo_local
                    mc = valid & (p >= 0) & (p < w)
                    psafe = jnp.where(mc, p, 0)
                    gather_fn(kv, psafe, mc)
                    return cc

                lax.fori_loop(0, HB // 16, grp, 0)

            def one_pass(h, _unused):
                hbase = h * HB
                pltpu.sync_copy(hp_hbm.at[pl.ds(wid * HSTRIDE + hbase, HB)],
                                hitp_v)
                pltpu.sync_copy(hb_hbm.at[pl.ds(wid * HSTRIDE + hbase, HB)],
                                hitb_v)

                total = nchunks * 8

                def src_slice(q):
                    ci = lax.shift_right_logical(q, 3)
                    a = lax.bitwise_and(q, 7)
                    clo = jnp.minimum(rlo + ci * CW, clamp)
                    return tT.at[pl.ds(pl.multiple_of(a * 8, 8), 8),
                                 pl.ds(pl.multiple_of(clo, 128), CW)]

                pltpu.async_copy(src_slice(0), chunk_v.at[0], semA)

                def chunk_octet(q, cc):
                    ci = lax.shift_right_logical(q, 3)
                    a = lax.bitwise_and(q, 7)
                    clo = jnp.minimum(rlo + ci * CW, clamp)
                    par = lax.bitwise_and(q, 1)

                    def body0():
                        def issue():
                            pltpu.async_copy(src_slice(q + 1),
                                             chunk_v.at[1], semB)
                        pl.when(q + 1 < total)(issue)
                        pltpu.make_async_copy(
                            src_slice(q), chunk_v.at[0], semA).wait()

                    def body1():
                        def issue():
                            pltpu.async_copy(src_slice(q + 1),
                                             chunk_v.at[0], semA)
                        pl.when(q + 1 < total)(issue)
                        pltpu.make_async_copy(
                            src_slice(q), chunk_v.at[1], semB).wait()

                    pl.when(par == 0)(body0)
                    pl.when(par == 1)(body1)

                    def gfn(kv, psafe, mc):
                        for jj in range(8):
                            x = plsc.load_gather(
                                chunk_v,
                                [jnp.broadcast_to(par, (16,)),
                                 jnp.broadcast_to(jj, (16,)), psafe],
                                mask=mc)
                            plsc.store_scatter(
                                rows_v,
                                [kv, jnp.broadcast_to(a * 8 + jj, (16,))],
                                x, mask=mc)

                    extract_groups(hbase, clo - rlo, CW, gfn)
                    return cc

                lax.fori_loop(0, total, chunk_octet, 0)

                @pl.when(wid == tail_owner)
                def _():
                    def gfn(kv, psafe, mc):
                        for j in range(E):
                            x = plsc.load_gather(
                                tail_v,
                                [psafe, jnp.broadcast_to(j, (16,))], mask=mc)
                            plsc.store_scatter(
                                rows_v,
                                [kv, jnp.broadcast_to(j, (16,))], x, mask=mc)

                    extract_groups(hbase, tail_off, tail_n, gfn)

                def sgrp(g, cc):
                    kv = iota16 + g * 16
                    valid = (kv + hbase) < nvec
                    bvec = jnp.where(valid, hitb_v[pl.ds(g * 16, 16)],
                                     DUMMY + iota16)
                    stage_v[lax.shift_right_logical(g, 3),
                            pl.ds(lax.bitwise_and(g, 7) * 16, 16)] = bvec
                    return cc

                lax.fori_loop(0, HB // 16, sgrp, 0)
                for q in range(HB // 128):
                    @pl.when(c == 0)
                    def _(q=q):
                        pltpu.sync_copy(rows_v.at[pl.ds(q * 128, 128)],
                                        out_a.at[stage_v.at[q]])

                    @pl.when(c == 1)
                    def _(q=q):
                        pltpu.sync_copy(rows_v.at[pl.ds(q * 128, 128)],
                                        out_b.at[stage_v.at[q]])
                return _unused

            lax.fori_loop(0, np_static, one_pass, 0)

        rlo_u = wid * USLAB_SM + jnp.minimum(wid, 4) * 128
        phase(utT, rlo_u, 15, 999936 - CW, hup, hub, 0,
              utail, 0, USLAB_BIG, UTAIL_N, ua, ub)
        rlo_m = wid * MSLAB_SM + jnp.minimum(wid, 13) * 128
        phase(mtT, rlo_m, 2, 99968 - CW, hmp, hmb, 1,
              mtail, 1, MSLAB_BIG, MTAIL_N, ma, mb)

    return scan_sc


def _mlp_body(ua_ref, ub_ref, ma_ref, mb_ref, uid_ref, mid_ref,
              w1a_ref, w1b_ref, b1_ref, w2_ref, b2_ref,
              w3_ref, b3_ref, out_ref):
    ua_hi = 16 * USLAB_SM + 4 * 128
    ma_hi = 16 * MSLAB_SM + 13 * 128
    use_a_u = (uid_ref[...] < ua_hi) | (uid_ref[...] >= UTAIL_LO)
    use_a_m = (mid_ref[...] < ma_hi) | (mid_ref[...] >= MTAIL_LO)
    ue = jnp.where(use_a_u, ua_ref[...][:, :E], ub_ref[...][:, :E])
    me = jnp.where(use_a_m, ma_ref[...][:, :E], mb_ref[...][:, :E])
    h = (jnp.dot(ue, w1a_ref[...], preferred_element_type=jnp.float32)
         + jnp.dot(me, w1b_ref[...], preferred_element_type=jnp.float32)
         + b1_ref[...])
    h = jnp.maximum(h, 0.0)
    h = jnp.dot(h, w2_ref[...], preferred_element_type=jnp.float32) + b2_ref[...]
    h = jnp.maximum(h, 0.0)
    o = jnp.dot(h, w3_ref[...], preferred_element_type=jnp.float32) + b3_ref[...]
    out_ref[...] = jax.nn.sigmoid(o)


BB = 2048


def _mlp_tc(ua, ub, ma, mb, uid2, mid2, w1a, w1b, b1, w2, b2, w3, b3):
    fixed = lambda i: (0, 0)
    emb = lambda i: (i, 0)
    return pl.pallas_call(
        _mlp_body,
        grid=(B // BB,),
        in_specs=[
            pl.BlockSpec((BB, 128), emb),
            pl.BlockSpec((BB, 128), emb),
            pl.BlockSpec((BB, 128), emb),
            pl.BlockSpec((BB, 128), emb),
            pl.BlockSpec((BB, 1), emb),
            pl.BlockSpec((BB, 1), emb),
            pl.BlockSpec((E, 64), fixed),
            pl.BlockSpec((E, 64), fixed),
            pl.BlockSpec((1, 64), fixed),
            pl.BlockSpec((64, 32), fixed),
            pl.BlockSpec((1, 32), fixed),
            pl.BlockSpec((32, 1), fixed),
            pl.BlockSpec((1, 1), fixed),
        ],
        out_specs=pl.BlockSpec((BB, 1), emb),
        out_shape=jax.ShapeDtypeStruct((B, 1), jnp.float32),
    )(ua, ub, ma, mb, uid2, mid2, w1a, w1b, b1, w2, b2, w3, b3)


def kernel(user_id, movie_id, user_table, movie_table, W1, b1, W2, b2, W3, b3):
    utT = user_table.T
    mtT = movie_table.T
    uid = user_id.astype(jnp.int32)
    mid = movie_id.astype(jnp.int32)
    utail = jnp.pad(user_table[UTAIL_LO:, :], ((0, 0), (0, 128 - E)))
    mtail = jnp.pad(movie_table[MTAIL_LO:, :], ((0, 0), (0, 128 - E)))
    hup, hub, hmp, hmb, cnts = _make_select_sc()(uid, mid)
    counts = cnts.reshape(2 * NW, 16)[:, 0]
    overflow = jnp.max(counts) > HB
    scan_args = (utT, mtT, utail, mtail, hup, hub, hmp, hmb, cnts)
    ua, ub, ma, mb = lax.cond(
        overflow,
        lambda: _make_scan_sc(26)(*scan_args),
        lambda: _make_scan_sc(1)(*scan_args),
    )
    return _mlp_tc(ua, ub, ma, mb,
                   uid.reshape(B, 1), mid.reshape(B, 1),
                   W1[:E], W1[E:], b1.reshape(1, 64),
                   W2, b2.reshape(1, 32), W3, b3.reshape(1, 1))


# speedup vs baseline: 2.0006x; 1.0379x over previous
import functools

import jax
import jax.numpy as jnp
from jax import lax
from jax.experimental import pallas as pl
from jax.experimental.pallas import tpu as pltpu
from jax.experimental.pallas import tpu_sc as plsc

B = 16384
E = 64
NC = 2
NS = 16
NW = NC * NS

USLAB_BIG = 245 * 128
USLAB_SM = 244 * 128
UTAIL_LO = 999936
UTAIL_N = 64
MSLAB_BIG = 25 * 128
MSLAB_SM = 24 * 128
MTAIL_LO = 99968
MTAIL_N = 32

PBITS = 15
PMASK = (1 << PBITS) - 1
HB = 640
HSTRIDE = 26 * HB
DUMMY = B
CW = 17 * 128


def _wid():
    return lax.axis_index("c") * NS + lax.axis_index("s")


def _mesh():
    return plsc.VectorSubcoreMesh(core_axis_name="c", subcore_axis_name="s")


@functools.cache
def _make_select_sc():
    @functools.partial(
        pl.kernel,
        mesh=_mesh(),
        out_type=[
            jax.ShapeDtypeStruct((NW * HSTRIDE,), jnp.int32),
            jax.ShapeDtypeStruct((NW * HSTRIDE,), jnp.int32),
            jax.ShapeDtypeStruct((NW * HSTRIDE,), jnp.int32),
            jax.ShapeDtypeStruct((NW * HSTRIDE,), jnp.int32),
            jax.ShapeDtypeStruct((2 * NW * 16,), jnp.int32),
        ],
        scratch_types=[
            pltpu.VMEM((B,), jnp.int32),
            pltpu.VMEM((B,), jnp.int32),
            pltpu.VMEM((B,), jnp.int32),
            pltpu.VMEM((16,), jnp.int32),
        ],
        compiler_params=pltpu.CompilerParams(needs_layout_passes=False),
    )
    def select_sc(user_id, movie_id, hup, hub, hmp, hmb, cnts,
                  hitp_v, hitb_v, idseg_v, cnt_v):
        wid = _wid()
        iota16 = lax.iota(jnp.int32, 16)

        def phase(ids_hbm, rlo, wbig, wsm, nbig,
                  tail_owner, tail_lo, tail_n, hp_out, hb_out, cslot):
            slab_w = jnp.where(wid < nbig, wbig, wsm)
            rhi = rlo + slab_w
            tail_off = wbig

            pltpu.sync_copy(ids_hbm, idseg_v)

            def seg(si, carry):
                def grp(g, carry2):
                    n3, nvec = carry2
                    v = idseg_v[pl.ds(si * 2048 + g * 16, 16)]
                    bvec = iota16 + (si * 2048 + g * 16)
                    m = (v >= rlo) & (v < rhi)
                    p = v - rlo
                    mx = ((wid == tail_owner) & (v >= tail_lo)
                          & (v < tail_lo + tail_n))
                    p = jnp.where(mx, v - tail_lo + tail_off, p)
                    m = m | mx
                    plsc.store_compressed(hitp_v.at[pl.ds(n3, 16)],
                                          p, mask=m)
                    plsc.store_compressed(hitb_v.at[pl.ds(n3, 16)],
                                          bvec, mask=m)
                    ca = plsc.all_reduce_population_count(m)
                    return (n3 + lax.reduce_max(ca, (0,)), nvec + ca)

                return lax.fori_loop(0, 128, grp, carry)

            _, nvec = lax.fori_loop(0, B // 2048, seg,
                                    (0, jnp.zeros((16,), jnp.int32)))
            cnt_v[pl.ds(0, 16)] = nvec
            pltpu.sync_copy(hitp_v, hp_out.at[pl.ds(wid * HSTRIDE, B)])
            pltpu.sync_copy(hitb_v, hb_out.at[pl.ds(wid * HSTRIDE, B)])
            pltpu.sync_copy(cnt_v,
                            cnts.at[pl.ds(cslot * NW * 16 + wid * 16, 16)])

        rlo_u = wid * USLAB_SM + jnp.minimum(wid, 4) * 128
        phase(user_id, rlo_u, USLAB_BIG, USLAB_SM, 4, 0,
              UTAIL_LO, UTAIL_N, hup, hub, 0)
        rlo_m = wid * MSLAB_SM + jnp.minimum(wid, 13) * 128
        phase(movie_id, rlo_m, MSLAB_BIG, MSLAB_SM, 13, 1,
              MTAIL_LO, MTAIL_N, hmp, hmb, 1)

    return select_sc


@functools.cache
def _make_scan_sc(np_static):
    @functools.partial(
        pl.kernel,
        mesh=_mesh(),
        out_type=[jax.ShapeDtypeStruct((B + 16, 128), jnp.float32)
                  for _ in range(4)],
        scratch_types=[
            pltpu.VMEM((2, 8, CW), jnp.float32),
            pltpu.VMEM((HB, 128), jnp.float32),
            pltpu.VMEM((HB,), jnp.int32),
            pltpu.VMEM((HB,), jnp.int32),
            pltpu.VMEM((HB // 128, 128), jnp.int32),
            pltpu.VMEM((UTAIL_N, 128), jnp.float32),
            pltpu.VMEM((16,), jnp.int32),
            pltpu.SemaphoreType.DMA,
            pltpu.SemaphoreType.DMA,
        ],
        compiler_params=pltpu.CompilerParams(needs_layout_passes=False),
    )
    def scan_sc(utT, mtT, utail, mtail, hup, hub, hmp, hmb, cnts,
                ua, ub, ma, mb,
                chunk_v, rows_v, hitp_v, hitb_v, stage_v, tail_v, cnt_v,
                semA, semB):
        c = lax.axis_index("c")
        wid = _wid()
        iota16 = lax.iota(jnp.int32, 16)

        def phase(tT, rlo, nchunks, clamp, hp_hbm, hb_hbm, cslot,
                  tail_hbm, tail_owner, tail_off, tail_n, out_a, out_b):
            pltpu.sync_copy(
                cnts.at[pl.ds(cslot * NW * 16 + wid * 16, 16)], cnt_v)
            nvec = cnt_v[pl.ds(0, 16)]

            @pl.when(wid == tail_owner)
            def _():
                pltpu.sync_copy(tail_hbm.at[pl.ds(0, tail_n)],
                                tail_v.at[pl.ds(0, tail_n)])

            def extract_groups(hbase, clo_local, w, gather_fn):
                def grp(g, cc):
                    kv = iota16 + g * 16
                    valid = (kv + hbase) < nvec
                    p = hitp_v[pl.ds(g * 16, 16)] - clo_local
                    mc = valid & (p >= 0) & (p < w)
                    psafe = jnp.where(mc, p, 0)
                    gather_fn(kv, psafe, mc)
                    return cc

                lax.fori_loop(0, HB // 16, grp, 0)

            def one_pass(h, _unused):
                hbase = h * HB
                pltpu.sync_copy(hp_hbm.at[pl.ds(wid * HSTRIDE + hbase, HB)],
                                hitp_v)
                pltpu.sync_copy(hb_hbm.at[pl.ds(wid * HSTRIDE + hbase, HB)],
                                hitb_v)

                total = nchunks * 8

                def src_slice(q):
                    ci = lax.shift_right_logical(q, 3)
                    a = lax.bitwise_and(q, 7)
                    clo = jnp.minimum(rlo + ci * CW, clamp)
                    return tT.at[pl.ds(pl.multiple_of(a * 8, 8), 8),
                                 pl.ds(pl.multiple_of(clo, 128), CW)]

                pltpu.async_copy(src_slice(0), chunk_v.at[0], semA)

                def chunk_octet(q, cc):
                    ci = lax.shift_right_logical(q, 3)
                    a = lax.bitwise_and(q, 7)
                    clo = jnp.minimum(rlo + ci * CW, clamp)
                    par = lax.bitwise_and(q, 1)

                    def body0():
                        def issue():
                            pltpu.async_copy(src_slice(q + 1),
                                             chunk_v.at[1], semB)
                        pl.when(q + 1 < total)(issue)
                        pltpu.make_async_copy(
                            src_slice(q), chunk_v.at[0], semA).wait()

                    def body1():
                        def issue():
                            pltpu.async_copy(src_slice(q + 1),
                                             chunk_v.at[0], semA)
                        pl.when(q + 1 < total)(issue)
                        pltpu.make_async_copy(
                            src_slice(q), chunk_v.at[1], semB).wait()

                    pl.when(par == 0)(body0)
                    pl.when(par == 1)(body1)

                    def gfn(kv, psafe, mc):
                        for jj in range(8):
                            x = plsc.load_gather(
                                chunk_v,
                                [jnp.broadcast_to(par, (16,)),
                                 jnp.broadcast_to(jj, (16,)), psafe],
                                mask=mc)
                            plsc.store_scatter(
                                rows_v,
                                [kv, jnp.broadcast_to(a * 8 + jj, (16,))],
                                x, mask=mc)

                    extract_groups(hbase, clo - rlo, CW, gfn)
                    return cc

                lax.fori_loop(0, total, chunk_octet, 0)

                @pl.when(wid == tail_owner)
                def _():
                    def gfn(kv, psafe, mc):
                        for j in range(E):
                            x = plsc.load_gather(
                                tail_v,
                                [psafe, jnp.broadcast_to(j, (16,))], mask=mc)
                            plsc.store_scatter(
                                rows_v,
                                [kv, jnp.broadcast_to(j, (16,))], x, mask=mc)

                    extract_groups(hbase, tail_off, tail_n, gfn)

                def sgrp(g, cc):
                    kv = iota16 + g * 16
                    valid = (kv + hbase) < nvec
                    bvec = jnp.where(valid, hitb_v[pl.ds(g * 16, 16)],
                                     DUMMY + iota16)
                    stage_v[lax.shift_right_logical(g, 3),
                            pl.ds(lax.bitwise_and(g, 7) * 16, 16)] = bvec
                    return cc

                lax.fori_loop(0, HB // 16, sgrp, 0)
                for q in range(HB // 128):
                    @pl.when(c == 0)
                    def _(q=q):
                        pltpu.sync_copy(rows_v.at[pl.ds(q * 128, 128)],
                                        out_a.at[stage_v.at[q]])

                    @pl.when(c == 1)
                    def _(q=q):
                        pltpu.sync_copy(rows_v.at[pl.ds(q * 128, 128)],
                                        out_b.at[stage_v.at[q]])
                return _unused

            lax.fori_loop(0, np_static, one_pass, 0)

        rlo_u = wid * USLAB_SM + jnp.minimum(wid, 4) * 128
        phase(utT, rlo_u, 15, 999936 - CW, hup, hub, 0,
              utail, 0, USLAB_BIG, UTAIL_N, ua, ub)
        rlo_m = wid * MSLAB_SM + jnp.minimum(wid, 13) * 128
        phase(mtT, rlo_m, 2, 99968 - CW, hmp, hmb, 1,
              mtail, 1, MSLAB_BIG, MTAIL_N, ma, mb)

    return scan_sc


def _mlp_body(ua_ref, ub_ref, ma_ref, mb_ref, uid_ref, mid_ref,
              w1a_ref, w1b_ref, b1_ref, w2_ref, b2_ref,
              w3_ref, b3_ref, out_ref):
    ua_hi = 16 * USLAB_SM + 4 * 128
    ma_hi = 16 * MSLAB_SM + 13 * 128
    use_a_u = (uid_ref[...] < ua_hi) | (uid_ref[...] >= UTAIL_LO)
    use_a_m = (mid_ref[...] < ma_hi) | (mid_ref[...] >= MTAIL_LO)
    ue = jnp.where(use_a_u, ua_ref[...][:, :E], ub_ref[...][:, :E])
    me = jnp.where(use_a_m, ma_ref[...][:, :E], mb_ref[...][:, :E])
    h = (jnp.dot(ue, w1a_ref[...], preferred_element_type=jnp.float32)
         + jnp.dot(me, w1b_ref[...], preferred_element_type=jnp.float32)
         + b1_ref[...])
    h = jnp.maximum(h, 0.0)
    h = jnp.dot(h, w2_ref[...], preferred_element_type=jnp.float32) + b2_ref[...]
    h = jnp.maximum(h, 0.0)
    o = jnp.dot(h, w3_ref[...], preferred_element_type=jnp.float32) + b3_ref[...]
    out_ref[...] = jax.nn.sigmoid(o)


BB = 4096


def _mlp_tc(ua, ub, ma, mb, uid2, mid2, w1a, w1b, b1, w2, b2, w3, b3):
    fixed = lambda i: (0, 0)
    emb = lambda i: (i, 0)
    return pl.pallas_call(
        _mlp_body,
        grid=(B // BB,),
        in_specs=[
            pl.BlockSpec((BB, 128), emb),
            pl.BlockSpec((BB, 128), emb),
            pl.BlockSpec((BB, 128), emb),
            pl.BlockSpec((BB, 128), emb),
            pl.BlockSpec((BB, 1), emb),
            pl.BlockSpec((BB, 1), emb),
            pl.BlockSpec((E, 64), fixed),
            pl.BlockSpec((E, 64), fixed),
            pl.BlockSpec((1, 64), fixed),
            pl.BlockSpec((64, 32), fixed),
            pl.BlockSpec((1, 32), fixed),
            pl.BlockSpec((32, 1), fixed),
            pl.BlockSpec((1, 1), fixed),
        ],
        out_specs=pl.BlockSpec((BB, 1), emb),
        out_shape=jax.ShapeDtypeStruct((B, 1), jnp.float32),
    )(ua, ub, ma, mb, uid2, mid2, w1a, w1b, b1, w2, b2, w3, b3)


def kernel(user_id, movie_id, user_table, movie_table, W1, b1, W2, b2, W3, b3):
    utT = user_table.T
    mtT = movie_table.T
    uid = user_id.astype(jnp.int32)
    mid = movie_id.astype(jnp.int32)
    utail = jnp.pad(user_table[UTAIL_LO:, :], ((0, 0), (0, 128 - E)))
    mtail = jnp.pad(movie_table[MTAIL_LO:, :], ((0, 0), (0, 128 - E)))
    hup, hub, hmp, hmb, cnts = _make_select_sc()(uid, mid)
    counts = cnts.reshape(2 * NW, 16)[:, 0]
    overflow = jnp.max(counts) > HB
    scan_args = (utT, mtT, utail, mtail, hup, hub, hmp, hmb, cnts)
    ua, ub, ma, mb = lax.cond(
        overflow,
        lambda: _make_scan_sc(26)(*scan_args),
        lambda: _make_scan_sc(1)(*scan_args),
    )
    return _mlp_tc(ua, ub, ma, mb,
                   uid.reshape(B, 1), mid.reshape(B, 1),
                   W1[:E], W1[E:], b1.reshape(1, 64),
                   W2, b2.reshape(1, 32), W3, b3.reshape(1, 1))
